# baseline (speedup 1.0000x reference)
import functools

import jax
import jax.numpy as jnp
from jax import lax
from jax.experimental import pallas as pl
from jax.experimental.pallas import tpu as pltpu
from jax.experimental.pallas import tpu_sc as plsc

E = 8
TOP_K = 2
H = 768
I = 2048
S = 2048

BLK = 256
NA = S * TOP_K
NROWS = NA + E * BLK
NB = NROWS // BLK
NC, NS = 2, 16
NW = NC * NS
RPW = NROWS // NW
GCH = 80
TPW = S // NW
CCH = 32


_DN = (((1,), (1,)), ((), ()))


def _router_body(x_ref, g_ref, rw_ref, sel_ref):
    logits = lax.dot_general(x_ref[...], g_ref[...], _DN,
                             preferred_element_type=jnp.float32)
    idx = lax.broadcasted_iota(jnp.int32, (S, E), 1)
    m1 = jnp.max(logits, axis=1, keepdims=True)
    a1 = jnp.min(jnp.where(logits == m1, idx, E), axis=1, keepdims=True)
    masked = jnp.where(idx == a1, -jnp.inf, logits)
    m2 = jnp.max(masked, axis=1, keepdims=True)
    a2 = jnp.min(jnp.where(masked == m2, idx, E), axis=1, keepdims=True)
    e2 = jnp.exp(m2 - m1)
    w0 = 1.0 / (1.0 + e2)
    rw_ref[...] = jnp.concatenate([w0, 1.0 - w0], axis=1)
    sel_ref[...] = jnp.concatenate([a1, a2], axis=1)


def _dispatch_plan(sel):
    flat_e = sel.reshape(NA)
    onehot = (flat_e[:, None] == jnp.arange(E, dtype=jnp.int32)[None, :])
    csum = jnp.cumsum(onehot.astype(jnp.int32), axis=0)
    counts = csum[-1]
    padded = ((counts + BLK - 1) // BLK) * BLK
    starts_pad = jnp.concatenate(
        [jnp.zeros((1,), jnp.int32), jnp.cumsum(padded).astype(jnp.int32)])[:E]
    rank = jnp.take_along_axis(csum, flat_e[:, None], axis=1)[:, 0] - 1
    rows = starts_pad[flat_e] + rank
    inv = rows.reshape(S, TOP_K)
    jblk = jnp.arange(NB, dtype=jnp.int32) * BLK
    be = (jnp.searchsorted(starts_pad, jblk, side="right") - 1).astype(jnp.int32)
    bvalid = (jblk < jnp.sum(padded)).astype(jnp.int32)
    inv3 = inv.reshape(NW, TPW, TOP_K).transpose(0, 2, 1)
    return inv, inv3, be, bvalid


def _expert_body(be_ref, bv_ref, x_ref, w1_ref, v_ref, w2_ref, y_ref):
    j = pl.program_id(0)

    @pl.when(bv_ref[j] == 1)
    def _():
        xb = x_ref[...]
        g = lax.dot_general(xb, w1_ref[0], _DN,
                            preferred_element_type=jnp.float32)
        sg = g * (1.0 / (1.0 + jnp.exp(-g)))
        vv = lax.dot_general(xb, v_ref[0], _DN,
                             preferred_element_type=jnp.float32)
        h = (sg * vv).astype(jnp.bfloat16)
        y_ref[...] = lax.dot_general(h, w2_ref[0], _DN,
                                     preferred_element_type=jnp.float32)

    @pl.when(bv_ref[j] == 0)
    def _():
        y_ref[...] = jnp.zeros((BLK, H), jnp.float32)


def _combine_body(ya_ref, yb_ref, rw_ref, o_ref):
    o_ref[...] = (ya_ref[...] * rw_ref[:, 0:1] + yb_ref[...] * rw_ref[:, 1:2])


@functools.lru_cache(maxsize=None)
def _sc_kernels():
    mesh = plsc.VectorSubcoreMesh(core_axis_name="c", subcore_axis_name="s",
                                  num_cores=NC, num_subcores=NS)

    @functools.partial(
        pl.kernel,
        out_type=jax.ShapeDtypeStruct((NROWS, H // 2), jnp.int32),
        mesh=mesh,
        scratch_types=[
            pltpu.VMEM((TOP_K, TPW), jnp.int32),
            pltpu.VMEM((TPW, H // 2), jnp.int32),
            pltpu.SemaphoreType.DMA,
        ],
    )
    def _sc_scatter_x(x_hbm, inv3_hbm, out_hbm, idx2, buf, wsem):
        wid = lax.axis_index("s") * NC + lax.axis_index("c")
        pltpu.sync_copy(inv3_hbm.at[wid], idx2)
        pltpu.sync_copy(x_hbm.at[pl.ds(wid * TPW, TPW)], buf)
        s0 = pltpu.async_copy(buf, out_hbm.at[idx2.at[0]], wsem)
        s1 = pltpu.async_copy(buf, out_hbm.at[idx2.at[1]], wsem)
        s0.wait()
        s1.wait()

    @functools.partial(
        pl.kernel,
        out_type=(jax.ShapeDtypeStruct((S, H), jnp.float32),
                  jax.ShapeDtypeStruct((S, H), jnp.float32)),
        mesh=mesh,
        scratch_types=[
            pltpu.VMEM((TPW,), jnp.int32),
            pltpu.VMEM((TPW,), jnp.int32),
            pltpu.VMEM((TPW, H), jnp.float32),
            pltpu.VMEM((TPW, H), jnp.float32),
            pltpu.SemaphoreType.DMA,
            pltpu.SemaphoreType.DMA,
        ],
    )
    def _sc_gather_y(y_hbm, ia_hbm, ib_hbm, oa_hbm, ob_hbm,
                     ia_v, ib_v, ra_v, rb_v, gsem, wsem):
        wid = lax.axis_index("s") * NC + lax.axis_index("c")
        base = wid * TPW
        gathers, writes = [], []
        for src, idx_v, buf in ((ia_hbm, ia_v, ra_v), (ib_hbm, ib_v, rb_v)):
            pltpu.sync_copy(src.at[pl.ds(base, TPW)], idx_v)
            gathers.append(pltpu.async_copy(y_hbm.at[idx_v], buf, gsem))
        for g, dst, buf in ((gathers[0], oa_hbm, ra_v), (gathers[1], ob_hbm, rb_v)):
            g.wait()
            writes.append(pltpu.async_copy(buf, dst.at[pl.ds(base, TPW)], wsem))
        for w in writes:
            w.wait()

    return _sc_scatter_x, _sc_gather_y


def _expert_call(x_disp, W1, V, W2, be, bvalid):
    grid_spec = pltpu.PrefetchScalarGridSpec(
        num_scalar_prefetch=2,
        grid=(NB,),
        in_specs=[
            pl.BlockSpec((BLK, H), lambda j, be_r, bv_r: (j, 0)),
            pl.BlockSpec((1, I, H), lambda j, be_r, bv_r: (be_r[j], 0, 0)),
            pl.BlockSpec((1, I, H), lambda j, be_r, bv_r: (be_r[j], 0, 0)),
            pl.BlockSpec((1, H, I), lambda j, be_r, bv_r: (be_r[j], 0, 0)),
        ],
        out_specs=pl.BlockSpec((BLK, H), lambda j, be_r, bv_r: (j, 0)),
    )
    return pl.pallas_call(
        _expert_body,
        grid_spec=grid_spec,
        out_shape=jax.ShapeDtypeStruct((NROWS, H), jnp.float32),
    )(be, bvalid, x_disp, W1, V, W2)


def _router_call(x, gate_w):
    return pl.pallas_call(
        _router_body,
        out_shape=(jax.ShapeDtypeStruct((S, TOP_K), jnp.float32),
                   jax.ShapeDtypeStruct((S, TOP_K), jnp.int32)),
    )(x, gate_w)


def _combine_call(ya, yb, rw):
    n = 16
    return pl.pallas_call(
        _combine_body,
        grid=(n,),
        in_specs=[
            pl.BlockSpec((S // n, H), lambda i: (i, 0)),
            pl.BlockSpec((S // n, H), lambda i: (i, 0)),
            pl.BlockSpec((S // n, TOP_K), lambda i: (i, 0)),
        ],
        out_specs=pl.BlockSpec((S // n, H), lambda i: (i, 0)),
        out_shape=jax.ShapeDtypeStruct((S, H), jnp.float32),
    )(ya, yb, rw)


def kernel(hidden_states, gate_w, W1, V, W2):
    x = hidden_states.reshape(S, H)
    w1b = W1.astype(jnp.bfloat16)
    vb = V.astype(jnp.bfloat16)
    w2b = W2.astype(jnp.bfloat16)
    xi = lax.bitcast_convert_type(
        x.astype(jnp.bfloat16).reshape(S, H // 2, 2), jnp.int32)
    rw, sel = _router_call(x, gate_w)
    inv, inv3, be, bvalid = _dispatch_plan(sel)
    sc_scatter_x, sc_gather_y = _sc_kernels()
    x_disp = lax.bitcast_convert_type(
        sc_scatter_x(xi, inv3), jnp.bfloat16).reshape(NROWS, H)
    y_disp = _expert_call(x_disp, w1b, vb, w2b, be, bvalid)
    ya, yb = sc_gather_y(y_disp, inv[:, 0], inv[:, 1])
    out = _combine_call(ya, yb, rw)
    b = hidden_states.shape[0]
    return (out.reshape(b, S, H), rw.reshape(b, S, TOP_K), sel.reshape(b, S, TOP_K))


# speedup vs baseline: 1.1770x; 1.1770x over previous
import functools

import jax
import jax.numpy as jnp
from jax import lax
from jax.experimental import pallas as pl
from jax.experimental.pallas import tpu as pltpu
from jax.experimental.pallas import tpu_sc as plsc

E = 8
TOP_K = 2
H = 768
I = 2048
S = 2048

BLK = 256
NA = S * TOP_K
NROWS = NA + E * BLK
NB = NROWS // BLK
NC, NS = 2, 16
NW = NC * NS
TPW = S // NW

_DN = (((1,), (1,)), ((), ()))


def _router_body(x_ref, g_ref, rw_ref, sel_ref):
    logits = lax.dot_general(x_ref[...], g_ref[...], _DN,
                             preferred_element_type=jnp.float32)
    idx = lax.broadcasted_iota(jnp.int32, (S, E), 1)
    m1 = jnp.max(logits, axis=1, keepdims=True)
    a1 = jnp.min(jnp.where(logits == m1, idx, E), axis=1, keepdims=True)
    masked = jnp.where(idx == a1, -jnp.inf, logits)
    m2 = jnp.max(masked, axis=1, keepdims=True)
    a2 = jnp.min(jnp.where(masked == m2, idx, E), axis=1, keepdims=True)
    e2 = jnp.exp(m2 - m1)
    w0 = 1.0 / (1.0 + e2)
    rw_ref[...] = jnp.concatenate([w0, 1.0 - w0], axis=1)
    sel_ref[...] = jnp.concatenate([a1, a2], axis=1)


def _dispatch_plan(sel):
    flat_e = sel.reshape(NA)
    onehot = (flat_e[:, None] == jnp.arange(E, dtype=jnp.int32)[None, :])
    csum = jnp.cumsum(onehot.astype(jnp.int32), axis=0)
    counts = csum[-1]
    padded = ((counts + BLK - 1) // BLK) * BLK
    starts_pad = jnp.concatenate(
        [jnp.zeros((1,), jnp.int32), jnp.cumsum(padded).astype(jnp.int32)])[:E]
    rank = jnp.take_along_axis(csum, flat_e[:, None], axis=1)[:, 0] - 1
    rows = starts_pad[flat_e] + rank
    inv = rows.reshape(S, TOP_K)
    jblk = jnp.arange(NB, dtype=jnp.int32) * BLK
    be = (jnp.searchsorted(starts_pad, jblk, side="right") - 1).astype(jnp.int32)
    bvalid = (jblk < jnp.sum(padded)).astype(jnp.int32)
    inv3 = inv.reshape(NW, TPW, TOP_K).transpose(0, 2, 1)
    return inv, inv3, be, bvalid


def _expert_body(be_ref, bv_ref, x_ref, w1_ref, v_ref, w2_ref, y_ref):
    j = pl.program_id(0)

    @pl.when(bv_ref[j] == 1)
    def _():
        xb = x_ref[...].astype(jnp.float32)
        g = lax.dot_general(xb, w1_ref[0], _DN,
                            preferred_element_type=jnp.float32)
        sg = g * (1.0 / (1.0 + jnp.exp(-g)))
        vv = lax.dot_general(xb, v_ref[0], _DN,
                             preferred_element_type=jnp.float32)
        y_ref[...] = lax.dot_general(sg * vv, w2_ref[0], _DN,
                                     preferred_element_type=jnp.float32)

    @pl.when(bv_ref[j] == 0)
    def _():
        y_ref[...] = jnp.zeros((BLK, H), jnp.float32)


@functools.lru_cache(maxsize=None)
def _sc_kernels():
    mesh = plsc.VectorSubcoreMesh(core_axis_name="c", subcore_axis_name="s",
                                  num_cores=NC, num_subcores=NS)

    @functools.partial(
        pl.kernel,
        out_type=jax.ShapeDtypeStruct((NROWS, H // 2), jnp.int32),
        mesh=mesh,
        scratch_types=[
            pltpu.VMEM((TOP_K, TPW), jnp.int32),
            pltpu.VMEM((TPW, H // 2), jnp.int32),
            pltpu.SemaphoreType.DMA,
        ],
    )
    def _sc_scatter_x(x_hbm, inv3_hbm, out_hbm, idx2, buf, wsem):
        wid = lax.axis_index("s") * NC + lax.axis_index("c")
        pltpu.sync_copy(inv3_hbm.at[wid], idx2)
        pltpu.sync_copy(x_hbm.at[pl.ds(wid * TPW, TPW)], buf)
        s0 = pltpu.async_copy(buf, out_hbm.at[idx2.at[0]], wsem)
        s1 = pltpu.async_copy(buf, out_hbm.at[idx2.at[1]], wsem)
        s0.wait()
        s1.wait()

    @functools.partial(
        pl.kernel,
        out_type=jax.ShapeDtypeStruct((S, H), jnp.float32),
        mesh=mesh,
        scratch_types=[
            pltpu.VMEM((TPW,), jnp.int32),
            pltpu.VMEM((TPW,), jnp.int32),
            pltpu.VMEM((TPW, 16), jnp.float32),
            pltpu.VMEM((TPW, 16), jnp.float32),
            pltpu.VMEM((TPW, H), jnp.float32),
            pltpu.VMEM((TPW, H), jnp.float32),
            pltpu.SemaphoreType.DMA,
        ],
    )
    def _sc_combine(y_hbm, ia_hbm, ib_hbm, wa_hbm, wb_hbm, out_hbm,
                    ia_v, ib_v, wa_v, wb_v, ra_v, rb_v, gsem):
        wid = lax.axis_index("s") * NC + lax.axis_index("c")
        base = wid * TPW
        pltpu.sync_copy(ia_hbm.at[pl.ds(base, TPW)], ia_v)
        pltpu.sync_copy(ib_hbm.at[pl.ds(base, TPW)], ib_v)
        pltpu.sync_copy(wa_hbm.at[pl.ds(base, TPW)], wa_v)
        pltpu.sync_copy(wb_hbm.at[pl.ds(base, TPW)], wb_v)
        ga = pltpu.async_copy(y_hbm.at[ia_v], ra_v, gsem)
        gb = pltpu.async_copy(y_hbm.at[ib_v], rb_v, gsem)
        ga.wait()
        gb.wait()

        def body(t, carry):
            was = wa_v[t, :]
            wbs = wb_v[t, :]
            for c in range(H // 16):
                sl = pl.ds(c * 16, 16)
                ra_v[t, sl] = ra_v[t, sl] * was + rb_v[t, sl] * wbs
            return carry

        lax.fori_loop(0, TPW, body, 0)
        pltpu.sync_copy(ra_v, out_hbm.at[pl.ds(base, TPW)])

    return _sc_scatter_x, _sc_combine


def _expert_call(x_disp, W1, V, W2, be, bvalid):
    grid_spec = pltpu.PrefetchScalarGridSpec(
        num_scalar_prefetch=2,
        grid=(NB,),
        in_specs=[
            pl.BlockSpec((BLK, H), lambda j, be_r, bv_r: (j, 0)),
            pl.BlockSpec((1, I, H), lambda j, be_r, bv_r: (be_r[j], 0, 0)),
            pl.BlockSpec((1, I, H), lambda j, be_r, bv_r: (be_r[j], 0, 0)),
            pl.BlockSpec((1, H, I), lambda j, be_r, bv_r: (be_r[j], 0, 0)),
        ],
        out_specs=pl.BlockSpec((BLK, H), lambda j, be_r, bv_r: (j, 0)),
    )
    return pl.pallas_call(
        _expert_body,
        grid_spec=grid_spec,
        out_shape=jax.ShapeDtypeStruct((NROWS, H), jnp.float32),
    )(be, bvalid, x_disp, W1, V, W2)


def _router_call(x, gate_w):
    return pl.pallas_call(
        _router_body,
        out_shape=(jax.ShapeDtypeStruct((S, TOP_K), jnp.float32),
                   jax.ShapeDtypeStruct((S, TOP_K), jnp.int32)),
    )(x, gate_w)


def kernel(hidden_states, gate_w, W1, V, W2):
    x = hidden_states.reshape(S, H)
    xi = lax.bitcast_convert_type(
        x.astype(jnp.bfloat16).reshape(S, H // 2, 2), jnp.int32)
    rw, sel = _router_call(x, gate_w)
    inv, inv3, be, bvalid = _dispatch_plan(sel)
    sc_scatter_x, sc_combine = _sc_kernels()
    x_disp = lax.bitcast_convert_type(
        sc_scatter_x(xi, inv3), jnp.bfloat16).reshape(NROWS, H)
    y_disp = _expert_call(x_disp, W1, V, W2, be, bvalid)
    wa2 = jnp.broadcast_to(rw[:, 0:1], (S, 16))
    wb2 = jnp.broadcast_to(rw[:, 1:2], (S, 16))
    out = sc_combine(y_disp, inv[:, 0], inv[:, 1], wa2, wb2)
    b = hidden_states.shape[0]
    return (out.reshape(b, S, H), rw.reshape(b, S, TOP_K), sel.reshape(b, S, TOP_K))


# speedup vs baseline: 2.0266x; 1.7218x over previous
import functools

import jax
import jax.numpy as jnp
from jax import lax
from jax.experimental import pallas as pl
from jax.experimental.pallas import tpu as pltpu
from jax.experimental.pallas import tpu_sc as plsc

E = 8
TOP_K = 2
H = 768
I = 2048
S = 2048

BLK = 256
NA = S * TOP_K
NROWS = NA + E * BLK
NB = NROWS // BLK
NC, NS = 2, 16
NW = NC * NS
RPW = NROWS // NW
GCH = 80
TPW = S // NW
CCH = 32


_DN = (((1,), (1,)), ((), ()))


def _router_body(x_ref, g_ref, rw_ref, sel_ref):
    logits = lax.dot_general(x_ref[...], g_ref[...], _DN,
                             preferred_element_type=jnp.float32)
    idx = lax.broadcasted_iota(jnp.int32, (S, E), 1)
    m1 = jnp.max(logits, axis=1, keepdims=True)
    a1 = jnp.min(jnp.where(logits == m1, idx, E), axis=1, keepdims=True)
    masked = jnp.where(idx == a1, -jnp.inf, logits)
    m2 = jnp.max(masked, axis=1, keepdims=True)
    a2 = jnp.min(jnp.where(masked == m2, idx, E), axis=1, keepdims=True)
    e2 = jnp.exp(m2 - m1)
    w0 = 1.0 / (1.0 + e2)
    rw_ref[...] = jnp.concatenate([w0, 1.0 - w0], axis=1)
    sel_ref[...] = jnp.concatenate([a1, a2], axis=1)


def _dispatch_plan(sel):
    flat_e = sel.reshape(NA)
    onehot = (flat_e[:, None] == jnp.arange(E, dtype=jnp.int32)[None, :])
    csum = jnp.cumsum(onehot.astype(jnp.int32), axis=0)
    counts = csum[-1]
    padded = ((counts + BLK - 1) // BLK) * BLK
    starts_pad = jnp.concatenate(
        [jnp.zeros((1,), jnp.int32), jnp.cumsum(padded).astype(jnp.int32)])[:E]
    rank = jnp.take_along_axis(csum, flat_e[:, None], axis=1)[:, 0] - 1
    rows = starts_pad[flat_e] + rank
    inv = rows.reshape(S, TOP_K)
    jblk = jnp.arange(NB, dtype=jnp.int32) * BLK
    be = (jnp.sum((starts_pad[None, :] <= jblk[:, None]).astype(jnp.int32),
                  axis=1) - 1).astype(jnp.int32)
    bvalid = (jblk < jnp.sum(padded)).astype(jnp.int32)
    inv3 = inv.reshape(NW, TPW, TOP_K).transpose(0, 2, 1)
    return inv, inv3, be, bvalid


def _expert_body(be_ref, bv_ref, x_ref, w1_ref, v_ref, w2_ref, y_ref):
    j = pl.program_id(0)

    @pl.when(bv_ref[j] == 1)
    def _():
        xb = x_ref[...]
        g = lax.dot_general(xb, w1_ref[0], _DN,
                            preferred_element_type=jnp.float32,
                            precision=lax.Precision.DEFAULT)
        sg = g * (1.0 / (1.0 + jnp.exp(-g)))
        vv = lax.dot_general(xb, v_ref[0], _DN,
                             preferred_element_type=jnp.float32,
                             precision=lax.Precision.DEFAULT)
        y_ref[...] = lax.dot_general(sg * vv, w2_ref[0], _DN,
                                     preferred_element_type=jnp.float32,
                                     precision=lax.Precision.DEFAULT)

    @pl.when(bv_ref[j] == 0)
    def _():
        y_ref[...] = jnp.zeros((BLK, H), jnp.float32)


def _combine_body(ya_ref, yb_ref, rw_ref, o_ref):
    o_ref[...] = (ya_ref[...] * rw_ref[:, 0:1] + yb_ref[...] * rw_ref[:, 1:2])


@functools.lru_cache(maxsize=None)
def _sc_kernels():
    mesh = plsc.VectorSubcoreMesh(core_axis_name="c", subcore_axis_name="s",
                                  num_cores=NC, num_subcores=NS)

    @functools.partial(
        pl.kernel,
        out_type=jax.ShapeDtypeStruct((NROWS, H), jnp.float32),
        mesh=mesh,
        scratch_types=[
            pltpu.VMEM((TOP_K, TPW), jnp.int32),
            pltpu.VMEM((TPW, H), jnp.float32),
            pltpu.SemaphoreType.DMA,
        ],
    )
    def _sc_scatter_x(x_hbm, inv3_hbm, out_hbm, idx2, buf, wsem):
        wid = lax.axis_index("s") * NC + lax.axis_index("c")
        pltpu.sync_copy(inv3_hbm.at[wid], idx2)
        pltpu.sync_copy(x_hbm.at[pl.ds(wid * TPW, TPW)], buf)
        s0 = pltpu.async_copy(buf, out_hbm.at[idx2.at[0]], wsem)
        s1 = pltpu.async_copy(buf, out_hbm.at[idx2.at[1]], wsem)
        s0.wait()
        s1.wait()

    @functools.partial(
        pl.kernel,
        out_type=(jax.ShapeDtypeStruct((S, H), jnp.float32),
                  jax.ShapeDtypeStruct((S, H), jnp.float32)),
        mesh=mesh,
        scratch_types=[
            pltpu.VMEM((TPW,), jnp.int32),
            pltpu.VMEM((TPW,), jnp.int32),
            pltpu.VMEM((TPW, H), jnp.float32),
            pltpu.VMEM((TPW, H), jnp.float32),
            pltpu.SemaphoreType.DMA,
            pltpu.SemaphoreType.DMA,
        ],
    )
    def _sc_gather_y(y_hbm, ia_hbm, ib_hbm, oa_hbm, ob_hbm,
                     ia_v, ib_v, ra_v, rb_v, gsem, wsem):
        wid = lax.axis_index("s") * NC + lax.axis_index("c")
        base = wid * TPW
        gathers, writes = [], []
        for src, idx_v, buf in ((ia_hbm, ia_v, ra_v), (ib_hbm, ib_v, rb_v)):
            pltpu.sync_copy(src.at[pl.ds(base, TPW)], idx_v)
            gathers.append(pltpu.async_copy(y_hbm.at[idx_v], buf, gsem))
        for g, dst, buf in ((gathers[0], oa_hbm, ra_v), (gathers[1], ob_hbm, rb_v)):
            g.wait()
            writes.append(pltpu.async_copy(buf, dst.at[pl.ds(base, TPW)], wsem))
        for w in writes:
            w.wait()

    return _sc_scatter_x, _sc_gather_y


def _expert_call(x_disp, W1, V, W2, be, bvalid):
    grid_spec = pltpu.PrefetchScalarGridSpec(
        num_scalar_prefetch=2,
        grid=(NB,),
        in_specs=[
            pl.BlockSpec((BLK, H), lambda j, be_r, bv_r: (j, 0)),
            pl.BlockSpec((1, I, H), lambda j, be_r, bv_r: (be_r[j], 0, 0)),
            pl.BlockSpec((1, I, H), lambda j, be_r, bv_r: (be_r[j], 0, 0)),
            pl.BlockSpec((1, H, I), lambda j, be_r, bv_r: (be_r[j], 0, 0)),
        ],
        out_specs=pl.BlockSpec((BLK, H), lambda j, be_r, bv_r: (j, 0)),
    )
    return pl.pallas_call(
        _expert_body,
        grid_spec=grid_spec,
        out_shape=jax.ShapeDtypeStruct((NROWS, H), jnp.float32),
    )(be, bvalid, x_disp, W1, V, W2)


def _router_call(x, gate_w):
    return pl.pallas_call(
        _router_body,
        out_shape=(jax.ShapeDtypeStruct((S, TOP_K), jnp.float32),
                   jax.ShapeDtypeStruct((S, TOP_K), jnp.int32)),
    )(x, gate_w)


def _combine_call(ya, yb, rw):
    n = 16
    return pl.pallas_call(
        _combine_body,
        grid=(n,),
        in_specs=[
            pl.BlockSpec((S // n, H), lambda i: (i, 0)),
            pl.BlockSpec((S // n, H), lambda i: (i, 0)),
            pl.BlockSpec((S // n, TOP_K), lambda i: (i, 0)),
        ],
        out_specs=pl.BlockSpec((S // n, H), lambda i: (i, 0)),
        out_shape=jax.ShapeDtypeStruct((S, H), jnp.float32),
    )(ya, yb, rw)


def kernel(hidden_states, gate_w, W1, V, W2):
    x = hidden_states.reshape(S, H)
    rw, sel = _router_call(x, gate_w)
    inv, inv3, be, bvalid = _dispatch_plan(sel)
    sc_scatter_x, sc_gather_y = _sc_kernels()
    x_disp = sc_scatter_x(x, inv3)
    y_disp = _expert_call(x_disp, W1, V, W2, be, bvalid)
    ya, yb = sc_gather_y(y_disp, inv[:, 0], inv[:, 1])
    out = _combine_call(ya, yb, rw)
    b = hidden_states.shape[0]
    return (out.reshape(b, S, H), rw.reshape(b, S, TOP_K), sel.reshape(b, S, TOP_K))
